# baseline (speedup 1.0000x reference)
import functools
import math

import jax
import jax.numpy as jnp
import numpy as np
from jax.experimental import pallas as pl
from jax.experimental.pallas import tpu as pltpu

_T = 2048
_C = 768
_NSH = 12
_NBR = 4
_H = 48
_DH = 64
_K = 12
_EPS = 1.1920929e-07
_NEG = -1e30
_QBLK = 256
_SCALE = _DH ** (-0.5)
_GATE = math.pi / math.sqrt(3.0)
_F32 = jnp.float32


def _rope_consts():
    half = _DH // 2
    inv_freq = 1.0 / (10000.0 ** (np.arange(0, _DH, 2).astype(np.float32) / _DH))
    t = np.arange(_T, dtype=np.float32)
    freqs = np.outer(t, inv_freq)
    ct = np.repeat(np.cos(freqs), 2, axis=1).astype(np.float32)
    st = np.repeat(np.sin(freqs), 2, axis=1).astype(np.float32)
    dm = np.zeros((_DH, _DH), np.float32)
    em = np.zeros((_DH, _DH), np.float32)
    for i in range(half):
        dm[2 * i, i] = 1.0
        dm[2 * i + 1, half + i] = 1.0
        em[2 * i, half + i] = 1.0
        em[2 * i + 1, i] = -1.0
    return ct, st, dm, em


_CT_NP, _ST_NP, _DM_NP, _EM_NP = _rope_consts()


def _proj_body(a_ref, x_ref, wq_ref, bq_ref, wk_ref, bk_ref, q_ref, k_ref):
    q_ref[...] = (
        jnp.dot(a_ref[...], wq_ref[...], preferred_element_type=_F32) + bq_ref[...]
    )
    k_ref[...] = (
        jnp.dot(x_ref[...], wk_ref[...], preferred_element_type=_F32) + bk_ref[...]
    )


def _headprep_body(q_ref, kb_ref, m_ref, ct_ref, st_ref, dm_ref, em_ref,
                   qh_ref, kh_ref):
    q = q_ref[0]
    kb = kb_ref[0]
    mm = m_ref[0]
    ct = ct_ref[...]
    st = st_ref[...]
    dm = dm_ref[...]
    em = em_ref[...]
    qn = q / jnp.sqrt(jnp.mean(q * q, axis=1, keepdims=True) + _EPS)
    qm = jnp.dot(qn, mm, preferred_element_type=_F32)
    qr = (jnp.dot(qm * ct, dm, preferred_element_type=_F32)
          + jnp.dot(qm * st, em, preferred_element_type=_F32))
    qh_ref[0] = qr * _SCALE
    km = jnp.dot(kb, mm, preferred_element_type=_F32)
    kh_ref[0] = (jnp.dot(km * ct, dm, preferred_element_type=_F32)
                 + jnp.dot(km * st, em, preferred_element_type=_F32))


def _attn_body(qh_ref, kh_ref, kv_ref, kvq_ref, sink_ref, vn_ref,
               fcw_ref, fcb_ref, pw_ref, pb_ref, out_ref, *, jblk, width):
    h = pl.program_id(0)
    q = qh_ref[0]
    k = kh_ref[0]
    kv = kv_ref[0]
    kvq = kvq_ref[0]
    c = sink_ref[h]

    s = jax.lax.dot_general(
        q, k, dimension_numbers=(((1,), (1,)), ((), ())),
        preferred_element_type=_F32)
    rows = jblk * _QBLK + jax.lax.broadcasted_iota(jnp.int32, (_QBLK, width), 0)
    cols = jax.lax.broadcasted_iota(jnp.int32, (_QBLK, width), 1)
    s = jnp.where(cols > rows, _NEG, s)

    m = jnp.max(s, axis=1, keepdims=True)
    m2 = jnp.maximum(m, c)
    p = jnp.exp(s - m2)
    psink = jnp.exp(c - m2)
    z = jnp.sum(p, axis=1, keepdims=True) + psink

    cur = p
    for _ in range(_K - 1):
        mx = jnp.max(cur, axis=1, keepdims=True)
        cur = jnp.where(cur >= mx, -1.0, cur)
    tau = jnp.max(cur, axis=1, keepdims=True)
    pm = jnp.where(p >= tau, p, 0.0)

    ot = jnp.dot(pm, kv, preferred_element_type=_F32)
    marker = (ot / z + kvq) * (1.0 / (_K + 1))

    h1 = jnp.dot(marker, fcw_ref[...], preferred_element_type=_F32) + fcb_ref[...]
    h2 = h1 * h1 + 0.75 * h1 * h1 * h1
    h2 = h2 / jnp.sqrt(jnp.mean(h2 * h2, axis=1, keepdims=True) + _EPS)
    g = h2 * jax.nn.sigmoid(_GATE * h2)
    v = jnp.dot(g, pw_ref[...], preferred_element_type=_F32) + pb_ref[...]

    out_ref[0] = v + (psink / z) * vn_ref[0, 0]


def _outproj_body(c_ref, w_ref, b_ref, y_ref):
    acc = jnp.broadcast_to(b_ref[...], (_QBLK, _C)).astype(_F32)
    for hh in range(_H):
        acc = acc + jnp.dot(c_ref[hh], w_ref[hh], preferred_element_type=_F32)
    y_ref[...] = acc


def kernel(A, X, WK_w, WK_b, WQ_w, WQ_b, wedge_A, wedge_bias, sink_scalars,
           v_nulls, fc_w, fc_b, proj_w, proj_b, WO_w, WO_b):
    a2 = A.reshape(_T, _C)
    x2 = X.reshape(_T, _C)
    nmt = _T // _QBLK

    q_all, k_base = pl.pallas_call(
        _proj_body,
        grid=(nmt,),
        in_specs=[
            pl.BlockSpec((_QBLK, _C), lambda i: (i, 0)),
            pl.BlockSpec((_QBLK, _C), lambda i: (i, 0)),
            pl.BlockSpec((_C, _C * _NBR), lambda i: (0, 0)),
            pl.BlockSpec((1, _C * _NBR), lambda i: (0, 0)),
            pl.BlockSpec((_C, _C), lambda i: (0, 0)),
            pl.BlockSpec((1, _C), lambda i: (0, 0)),
        ],
        out_specs=[
            pl.BlockSpec((_QBLK, _C * _NBR), lambda i: (i, 0)),
            pl.BlockSpec((_QBLK, _C), lambda i: (i, 0)),
        ],
        out_shape=[
            jax.ShapeDtypeStruct((_T, _C * _NBR), _F32),
            jax.ShapeDtypeStruct((_T, _C), _F32),
        ],
    )(a2, x2, WQ_w, WQ_b.reshape(1, -1), WK_w, WK_b.reshape(1, -1))

    skew = wedge_A - wedge_A.T
    ms = (jnp.eye(_DH, dtype=_F32)[None]
          + skew[None]
          + jax.vmap(jnp.diag)(wedge_bias))
    ct = jnp.asarray(_CT_NP)
    st = jnp.asarray(_ST_NP)
    dm = jnp.asarray(_DM_NP)
    em = jnp.asarray(_EM_NP)

    q3 = q_all.reshape(_T, _H, _DH).transpose(1, 0, 2)
    kvan = k_base.reshape(_T, _NSH, _DH).transpose(1, 0, 2)

    qhat, khat = pl.pallas_call(
        _headprep_body,
        grid=(_H,),
        in_specs=[
            pl.BlockSpec((1, _T, _DH), lambda h: (h, 0, 0)),
            pl.BlockSpec((1, _T, _DH), lambda h: (h % _NSH, 0, 0)),
            pl.BlockSpec((1, _DH, _DH), lambda h: (h, 0, 0)),
            pl.BlockSpec((_T, _DH), lambda h: (0, 0)),
            pl.BlockSpec((_T, _DH), lambda h: (0, 0)),
            pl.BlockSpec((_DH, _DH), lambda h: (0, 0)),
            pl.BlockSpec((_DH, _DH), lambda h: (0, 0)),
        ],
        out_specs=[
            pl.BlockSpec((1, _T, _DH), lambda h: (h, 0, 0)),
            pl.BlockSpec((1, _T, _DH), lambda h: (h, 0, 0)),
        ],
        out_shape=[
            jax.ShapeDtypeStruct((_H, _T, _DH), _F32),
            jax.ShapeDtypeStruct((_H, _T, _DH), _F32),
        ],
    )(q3, kvan, ms, ct, st, dm, em)

    sink = sink_scalars.reshape(_H)
    vn3 = v_nulls.reshape(_H, 1, _DH)

    ctx_blocks = []
    for j in range(nmt):
        width = (j + 1) * _QBLK
        ctx_blocks.append(pl.pallas_call(
            functools.partial(_attn_body, jblk=j, width=width),
            grid=(_H,),
            in_specs=[
                pl.BlockSpec((1, _QBLK, _DH), lambda h, j=j: (h, j, 0)),
                pl.BlockSpec((1, width, _DH), lambda h: (h, 0, 0)),
                pl.BlockSpec((1, width, _DH), lambda h: (h % _NSH, 0, 0)),
                pl.BlockSpec((1, _QBLK, _DH), lambda h, j=j: (h % _NSH, j, 0)),
                pl.BlockSpec(memory_space=pltpu.SMEM),
                pl.BlockSpec((1, 1, _DH), lambda h: (h, 0, 0)),
                pl.BlockSpec((_DH, 4 * _DH), lambda h: (0, 0)),
                pl.BlockSpec((1, 4 * _DH), lambda h: (0, 0)),
                pl.BlockSpec((4 * _DH, _DH), lambda h: (0, 0)),
                pl.BlockSpec((1, _DH), lambda h: (0, 0)),
            ],
            out_specs=pl.BlockSpec((1, _QBLK, _DH), lambda h: (h, 0, 0)),
            out_shape=jax.ShapeDtypeStruct((_H, _QBLK, _DH), _F32),
        )(qhat, khat, kvan, kvan, sink, vn3,
          fc_w, fc_b.reshape(1, -1), proj_w, proj_b.reshape(1, -1)))
    ctx = jnp.concatenate(ctx_blocks, axis=1)

    wstack = WO_w.reshape(_NBR * _C, _C).reshape(_H, _DH, _C) * (1.0 / _NBR)
    bstack = jnp.mean(WO_b, axis=0).reshape(1, _C)

    y = pl.pallas_call(
        _outproj_body,
        grid=(nmt,),
        in_specs=[
            pl.BlockSpec((_H, _QBLK, _DH), lambda i: (0, i, 0)),
            pl.BlockSpec((_H, _DH, _C), lambda i: (0, 0, 0)),
            pl.BlockSpec((1, _C), lambda i: (0, 0)),
        ],
        out_specs=pl.BlockSpec((_QBLK, _C), lambda i: (i, 0)),
        out_shape=jax.ShapeDtypeStruct((_T, _C), _F32),
    )(ctx, wstack, bstack)

    return y.reshape(1, _T, _C)


# speedup vs baseline: 1.1481x; 1.1481x over previous
import functools
import math

import jax
import jax.numpy as jnp
import numpy as np
from jax.experimental import pallas as pl
from jax.experimental.pallas import tpu as pltpu

_T = 2048
_C = 768
_NSH = 12
_NBR = 4
_H = 48
_DH = 64
_K = 12
_EPS = 1.1920929e-07
_NEG = -1e30
_QBLK = 512
_SCALE = _DH ** (-0.5)
_GATE = math.pi / math.sqrt(3.0)
_F32 = jnp.float32


def _rope_consts():
    half = _DH // 2
    inv_freq = 1.0 / (10000.0 ** (np.arange(0, _DH, 2).astype(np.float32) / _DH))
    t = np.arange(_T, dtype=np.float32)
    freqs = np.outer(t, inv_freq)
    ct = np.repeat(np.cos(freqs), 2, axis=1).astype(np.float32)
    st = np.repeat(np.sin(freqs), 2, axis=1).astype(np.float32)
    dm = np.zeros((_DH, _DH), np.float32)
    em = np.zeros((_DH, _DH), np.float32)
    for i in range(half):
        dm[2 * i, i] = 1.0
        dm[2 * i + 1, half + i] = 1.0
        em[2 * i, half + i] = 1.0
        em[2 * i + 1, i] = -1.0
    return ct, st, dm, em


_CT_NP, _ST_NP, _DM_NP, _EM_NP = _rope_consts()


def _proj_body(a_ref, x_ref, wq_ref, bq_ref, wk_ref, bk_ref, q_ref, k_ref):
    q_ref[...] = (
        jnp.dot(a_ref[...], wq_ref[...], preferred_element_type=_F32) + bq_ref[...]
    )
    k_ref[...] = (
        jnp.dot(x_ref[...], wk_ref[...], preferred_element_type=_F32) + bk_ref[...]
    )


def _headprep_body(q_ref, kb_ref, m_ref, ct_ref, st_ref, dm_ref, em_ref,
                   qh_ref, kh_ref):
    q = q_ref[0]
    kb = kb_ref[0]
    mm = m_ref[0]
    ct = ct_ref[...]
    st = st_ref[...]
    dm = dm_ref[...]
    em = em_ref[...]
    qn = q / jnp.sqrt(jnp.mean(q * q, axis=1, keepdims=True) + _EPS)
    qm = jnp.dot(qn, mm, preferred_element_type=_F32)
    qr = (jnp.dot(qm * ct, dm, preferred_element_type=_F32)
          + jnp.dot(qm * st, em, preferred_element_type=_F32))
    qh_ref[0] = qr * _SCALE
    km = jnp.dot(kb, mm, preferred_element_type=_F32)
    kh_ref[0] = (jnp.dot(km * ct, dm, preferred_element_type=_F32)
                 + jnp.dot(km * st, em, preferred_element_type=_F32))


def _attn_body(qh_ref, kh_ref, kv_ref, kvq_ref, sink_ref, vn_ref,
               fcw_ref, fcb_ref, pw_ref, pb_ref, out_ref, *, jblk, width):
    h = pl.program_id(0)
    q = qh_ref[0]
    k = kh_ref[0]
    kv = kv_ref[0]
    kvq = kvq_ref[0]
    c = sink_ref[h]

    s = jax.lax.dot_general(
        q, k, dimension_numbers=(((1,), (1,)), ((), ())),
        preferred_element_type=_F32)
    rows = jblk * _QBLK + jax.lax.broadcasted_iota(jnp.int32, (_QBLK, width), 0)
    cols = jax.lax.broadcasted_iota(jnp.int32, (_QBLK, width), 1)
    s = jnp.where(cols > rows, _NEG, s)

    m = jnp.max(s, axis=1, keepdims=True)
    m2 = jnp.maximum(m, c)
    p = jnp.exp(s - m2)
    psink = jnp.exp(c - m2)
    z = jnp.sum(p, axis=1, keepdims=True) + psink

    cur = p
    for _ in range(_K - 1):
        mx = jnp.max(cur, axis=1, keepdims=True)
        cur = jnp.where(cur >= mx, -1.0, cur)
    tau = jnp.max(cur, axis=1, keepdims=True)
    pm = jnp.where(p >= tau, p, 0.0)

    ot = jnp.dot(pm, kv, preferred_element_type=_F32)
    marker = (ot / z + kvq) * (1.0 / (_K + 1))

    h1 = jnp.dot(marker, fcw_ref[...], preferred_element_type=_F32) + fcb_ref[...]
    h2 = h1 * h1 + 0.75 * h1 * h1 * h1
    h2 = h2 / jnp.sqrt(jnp.mean(h2 * h2, axis=1, keepdims=True) + _EPS)
    g = h2 * jax.nn.sigmoid(_GATE * h2)
    v = jnp.dot(g, pw_ref[...], preferred_element_type=_F32) + pb_ref[...]

    out_ref[0] = v + (psink / z) * vn_ref[0, 0]


def _outproj_body(c_ref, w_ref, b_ref, y_ref):
    acc = jnp.broadcast_to(b_ref[...], (_QBLK, _C)).astype(_F32)
    for hh in range(_H):
        acc = acc + jnp.dot(c_ref[hh], w_ref[hh], preferred_element_type=_F32)
    y_ref[...] = acc


def kernel(A, X, WK_w, WK_b, WQ_w, WQ_b, wedge_A, wedge_bias, sink_scalars,
           v_nulls, fc_w, fc_b, proj_w, proj_b, WO_w, WO_b):
    a2 = A.reshape(_T, _C)
    x2 = X.reshape(_T, _C)
    nmt = _T // _QBLK

    q_all, k_base = pl.pallas_call(
        _proj_body,
        grid=(nmt,),
        in_specs=[
            pl.BlockSpec((_QBLK, _C), lambda i: (i, 0)),
            pl.BlockSpec((_QBLK, _C), lambda i: (i, 0)),
            pl.BlockSpec((_C, _C * _NBR), lambda i: (0, 0)),
            pl.BlockSpec((1, _C * _NBR), lambda i: (0, 0)),
            pl.BlockSpec((_C, _C), lambda i: (0, 0)),
            pl.BlockSpec((1, _C), lambda i: (0, 0)),
        ],
        out_specs=[
            pl.BlockSpec((_QBLK, _C * _NBR), lambda i: (i, 0)),
            pl.BlockSpec((_QBLK, _C), lambda i: (i, 0)),
        ],
        out_shape=[
            jax.ShapeDtypeStruct((_T, _C * _NBR), _F32),
            jax.ShapeDtypeStruct((_T, _C), _F32),
        ],
    )(a2, x2, WQ_w, WQ_b.reshape(1, -1), WK_w, WK_b.reshape(1, -1))

    skew = wedge_A - wedge_A.T
    ms = (jnp.eye(_DH, dtype=_F32)[None]
          + skew[None]
          + jax.vmap(jnp.diag)(wedge_bias))
    ct = jnp.asarray(_CT_NP)
    st = jnp.asarray(_ST_NP)
    dm = jnp.asarray(_DM_NP)
    em = jnp.asarray(_EM_NP)

    q3 = q_all.reshape(_T, _H, _DH).transpose(1, 0, 2)
    kvan = k_base.reshape(_T, _NSH, _DH).transpose(1, 0, 2)

    qhat, khat = pl.pallas_call(
        _headprep_body,
        grid=(_H,),
        in_specs=[
            pl.BlockSpec((1, _T, _DH), lambda h: (h, 0, 0)),
            pl.BlockSpec((1, _T, _DH), lambda h: (h % _NSH, 0, 0)),
            pl.BlockSpec((1, _DH, _DH), lambda h: (h, 0, 0)),
            pl.BlockSpec((_T, _DH), lambda h: (0, 0)),
            pl.BlockSpec((_T, _DH), lambda h: (0, 0)),
            pl.BlockSpec((_DH, _DH), lambda h: (0, 0)),
            pl.BlockSpec((_DH, _DH), lambda h: (0, 0)),
        ],
        out_specs=[
            pl.BlockSpec((1, _T, _DH), lambda h: (h, 0, 0)),
            pl.BlockSpec((1, _T, _DH), lambda h: (h, 0, 0)),
        ],
        out_shape=[
            jax.ShapeDtypeStruct((_H, _T, _DH), _F32),
            jax.ShapeDtypeStruct((_H, _T, _DH), _F32),
        ],
    )(q3, kvan, ms, ct, st, dm, em)

    sink = sink_scalars.reshape(_H)
    vn3 = v_nulls.reshape(_H, 1, _DH)

    ctx_blocks = []
    for j in range(nmt):
        width = (j + 1) * _QBLK
        ctx_blocks.append(pl.pallas_call(
            functools.partial(_attn_body, jblk=j, width=width),
            grid=(_H,),
            in_specs=[
                pl.BlockSpec((1, _QBLK, _DH), lambda h, j=j: (h, j, 0)),
                pl.BlockSpec((1, width, _DH), lambda h: (h, 0, 0)),
                pl.BlockSpec((1, width, _DH), lambda h: (h % _NSH, 0, 0)),
                pl.BlockSpec((1, _QBLK, _DH), lambda h, j=j: (h % _NSH, j, 0)),
                pl.BlockSpec(memory_space=pltpu.SMEM),
                pl.BlockSpec((1, 1, _DH), lambda h: (h, 0, 0)),
                pl.BlockSpec((_DH, 4 * _DH), lambda h: (0, 0)),
                pl.BlockSpec((1, 4 * _DH), lambda h: (0, 0)),
                pl.BlockSpec((4 * _DH, _DH), lambda h: (0, 0)),
                pl.BlockSpec((1, _DH), lambda h: (0, 0)),
            ],
            out_specs=pl.BlockSpec((1, _QBLK, _DH), lambda h: (h, 0, 0)),
            out_shape=jax.ShapeDtypeStruct((_H, _QBLK, _DH), _F32),
        )(qhat, khat, kvan, kvan, sink, vn3,
          fc_w, fc_b.reshape(1, -1), proj_w, proj_b.reshape(1, -1)))
    ctx = jnp.concatenate(ctx_blocks, axis=1)

    wstack = WO_w.reshape(_NBR * _C, _C).reshape(_H, _DH, _C) * (1.0 / _NBR)
    bstack = jnp.mean(WO_b, axis=0).reshape(1, _C)

    y = pl.pallas_call(
        _outproj_body,
        grid=(nmt,),
        in_specs=[
            pl.BlockSpec((_H, _QBLK, _DH), lambda i: (0, i, 0)),
            pl.BlockSpec((_H, _DH, _C), lambda i: (0, 0, 0)),
            pl.BlockSpec((1, _C), lambda i: (0, 0)),
        ],
        out_specs=pl.BlockSpec((_QBLK, _C), lambda i: (i, 0)),
        out_shape=jax.ShapeDtypeStruct((_T, _C), _F32),
    )(ctx, wstack, bstack)

    return y.reshape(1, _T, _C)


# speedup vs baseline: 1.3803x; 1.2022x over previous
import functools
import math

import jax
import jax.numpy as jnp
import numpy as np
from jax.experimental import pallas as pl
from jax.experimental.pallas import tpu as pltpu

_T = 2048
_C = 768
_NSH = 12
_NBR = 4
_H = 48
_DH = 64
_K = 12
_EPS = 1.1920929e-07
_NEG = -1e30
_QBLK = 512
_SCALE = _DH ** (-0.5)
_GATE = math.pi / math.sqrt(3.0)
_F32 = jnp.float32


def _rope_consts():
    half = _DH // 2
    inv_freq = 1.0 / (10000.0 ** (np.arange(0, _DH, 2).astype(np.float32) / _DH))
    t = np.arange(_T, dtype=np.float32)
    freqs = np.outer(t, inv_freq)
    ct = np.repeat(np.cos(freqs), 2, axis=1).astype(np.float32)
    st = np.repeat(np.sin(freqs), 2, axis=1).astype(np.float32)
    dm = np.zeros((_DH, _DH), np.float32)
    em = np.zeros((_DH, _DH), np.float32)
    for i in range(half):
        dm[2 * i, i] = 1.0
        dm[2 * i + 1, half + i] = 1.0
        em[2 * i, half + i] = 1.0
        em[2 * i + 1, i] = -1.0
    return ct, st, dm, em


_CT_NP, _ST_NP, _DM_NP, _EM_NP = _rope_consts()


def _proj_body(a_ref, x_ref, wq_ref, bq_ref, wk_ref, bk_ref, q_ref, k_ref):
    q_ref[...] = (
        jnp.dot(a_ref[...], wq_ref[...], preferred_element_type=_F32) + bq_ref[...]
    )
    k_ref[...] = (
        jnp.dot(x_ref[...], wk_ref[...], preferred_element_type=_F32) + bk_ref[...]
    )


def _headprep_body(q_ref, kb_ref, m_ref, ct_ref, st_ref, dm_ref, em_ref,
                   qh_ref, kh_ref):
    q = q_ref[0]
    kb = kb_ref[0]
    mm = m_ref[0]
    ct = ct_ref[...]
    st = st_ref[...]
    dm = dm_ref[...]
    em = em_ref[...]
    qn = q / jnp.sqrt(jnp.mean(q * q, axis=1, keepdims=True) + _EPS)
    qm = jnp.dot(qn, mm, preferred_element_type=_F32)
    qr = (jnp.dot(qm * ct, dm, preferred_element_type=_F32)
          + jnp.dot(qm * st, em, preferred_element_type=_F32))
    qh_ref[0] = qr * _SCALE
    km = jnp.dot(kb, mm, preferred_element_type=_F32)
    kh_ref[0] = (jnp.dot(km * ct, dm, preferred_element_type=_F32)
                 + jnp.dot(km * st, em, preferred_element_type=_F32))


def _attn_body(qh_ref, kh_ref, kv_ref, kvq_ref, sink_ref, vn_ref,
               fcw_ref, fcb_ref, pw_ref, pb_ref, out_ref, *, jblk, width):
    h = pl.program_id(0)
    q = qh_ref[0]
    k = kh_ref[0]
    kv = kv_ref[0]
    kvq = kvq_ref[0]
    c = sink_ref[h]

    s = jax.lax.dot_general(
        q, k, dimension_numbers=(((1,), (1,)), ((), ())),
        preferred_element_type=_F32)
    rows = jblk * _QBLK + jax.lax.broadcasted_iota(jnp.int32, (_QBLK, width), 0)
    cols = jax.lax.broadcasted_iota(jnp.int32, (_QBLK, width), 1)
    s = jnp.where(cols > rows, _NEG, s)

    m = jnp.max(s, axis=1, keepdims=True)
    m2 = jnp.maximum(m, c)
    p = jnp.exp(s - m2)
    psink = jnp.exp(c - m2)
    z = jnp.sum(p, axis=1, keepdims=True) + psink

    pb = p.astype(jnp.bfloat16)
    cur = pb
    for _ in range(_K - 1):
        mx = jnp.max(cur, axis=1, keepdims=True)
        cur = jnp.where(cur >= mx, jnp.bfloat16(-1.0), cur)
    tau = jnp.max(cur, axis=1, keepdims=True)
    pm = jnp.where(pb >= tau, p, 0.0)

    ot = jnp.dot(pm, kv, preferred_element_type=_F32)
    marker = (ot / z + kvq) * (1.0 / (_K + 1))

    h1 = jnp.dot(marker, fcw_ref[...], preferred_element_type=_F32) + fcb_ref[...]
    h2 = h1 * h1 + 0.75 * h1 * h1 * h1
    h2 = h2 / jnp.sqrt(jnp.mean(h2 * h2, axis=1, keepdims=True) + _EPS)
    g = h2 * jax.nn.sigmoid(_GATE * h2)
    v = jnp.dot(g, pw_ref[...], preferred_element_type=_F32) + pb_ref[...]

    out_ref[0] = v + (psink / z) * vn_ref[0, 0]


def _outproj_body(c_ref, w_ref, b_ref, y_ref):
    acc = jnp.broadcast_to(b_ref[...], (_QBLK, _C)).astype(_F32)
    for hh in range(_H):
        acc = acc + jnp.dot(c_ref[hh], w_ref[hh], preferred_element_type=_F32)
    y_ref[...] = acc


def kernel(A, X, WK_w, WK_b, WQ_w, WQ_b, wedge_A, wedge_bias, sink_scalars,
           v_nulls, fc_w, fc_b, proj_w, proj_b, WO_w, WO_b):
    a2 = A.reshape(_T, _C)
    x2 = X.reshape(_T, _C)
    nmt = _T // _QBLK

    q_all, k_base = pl.pallas_call(
        _proj_body,
        grid=(nmt,),
        in_specs=[
            pl.BlockSpec((_QBLK, _C), lambda i: (i, 0)),
            pl.BlockSpec((_QBLK, _C), lambda i: (i, 0)),
            pl.BlockSpec((_C, _C * _NBR), lambda i: (0, 0)),
            pl.BlockSpec((1, _C * _NBR), lambda i: (0, 0)),
            pl.BlockSpec((_C, _C), lambda i: (0, 0)),
            pl.BlockSpec((1, _C), lambda i: (0, 0)),
        ],
        out_specs=[
            pl.BlockSpec((_QBLK, _C * _NBR), lambda i: (i, 0)),
            pl.BlockSpec((_QBLK, _C), lambda i: (i, 0)),
        ],
        out_shape=[
            jax.ShapeDtypeStruct((_T, _C * _NBR), _F32),
            jax.ShapeDtypeStruct((_T, _C), _F32),
        ],
    )(a2, x2, WQ_w, WQ_b.reshape(1, -1), WK_w, WK_b.reshape(1, -1))

    skew = wedge_A - wedge_A.T
    ms = (jnp.eye(_DH, dtype=_F32)[None]
          + skew[None]
          + jax.vmap(jnp.diag)(wedge_bias))
    ct = jnp.asarray(_CT_NP)
    st = jnp.asarray(_ST_NP)
    dm = jnp.asarray(_DM_NP)
    em = jnp.asarray(_EM_NP)

    q3 = q_all.reshape(_T, _H, _DH).transpose(1, 0, 2)
    kvan = k_base.reshape(_T, _NSH, _DH).transpose(1, 0, 2)

    qhat, khat = pl.pallas_call(
        _headprep_body,
        grid=(_H,),
        in_specs=[
            pl.BlockSpec((1, _T, _DH), lambda h: (h, 0, 0)),
            pl.BlockSpec((1, _T, _DH), lambda h: (h % _NSH, 0, 0)),
            pl.BlockSpec((1, _DH, _DH), lambda h: (h, 0, 0)),
            pl.BlockSpec((_T, _DH), lambda h: (0, 0)),
            pl.BlockSpec((_T, _DH), lambda h: (0, 0)),
            pl.BlockSpec((_DH, _DH), lambda h: (0, 0)),
            pl.BlockSpec((_DH, _DH), lambda h: (0, 0)),
        ],
        out_specs=[
            pl.BlockSpec((1, _T, _DH), lambda h: (h, 0, 0)),
            pl.BlockSpec((1, _T, _DH), lambda h: (h, 0, 0)),
        ],
        out_shape=[
            jax.ShapeDtypeStruct((_H, _T, _DH), _F32),
            jax.ShapeDtypeStruct((_H, _T, _DH), _F32),
        ],
    )(q3, kvan, ms, ct, st, dm, em)

    sink = sink_scalars.reshape(_H)
    vn3 = v_nulls.reshape(_H, 1, _DH)

    ctx_blocks = []
    for j in range(nmt):
        width = (j + 1) * _QBLK
        ctx_blocks.append(pl.pallas_call(
            functools.partial(_attn_body, jblk=j, width=width),
            grid=(_H,),
            in_specs=[
                pl.BlockSpec((1, _QBLK, _DH), lambda h, j=j: (h, j, 0)),
                pl.BlockSpec((1, width, _DH), lambda h: (h, 0, 0)),
                pl.BlockSpec((1, width, _DH), lambda h: (h % _NSH, 0, 0)),
                pl.BlockSpec((1, _QBLK, _DH), lambda h, j=j: (h % _NSH, j, 0)),
                pl.BlockSpec(memory_space=pltpu.SMEM),
                pl.BlockSpec((1, 1, _DH), lambda h: (h, 0, 0)),
                pl.BlockSpec((_DH, 4 * _DH), lambda h: (0, 0)),
                pl.BlockSpec((1, 4 * _DH), lambda h: (0, 0)),
                pl.BlockSpec((4 * _DH, _DH), lambda h: (0, 0)),
                pl.BlockSpec((1, _DH), lambda h: (0, 0)),
            ],
            out_specs=pl.BlockSpec((1, _QBLK, _DH), lambda h: (h, 0, 0)),
            out_shape=jax.ShapeDtypeStruct((_H, _QBLK, _DH), _F32),
        )(qhat, khat, kvan, kvan, sink, vn3,
          fc_w, fc_b.reshape(1, -1), proj_w, proj_b.reshape(1, -1)))
    ctx = jnp.concatenate(ctx_blocks, axis=1)

    wstack = WO_w.reshape(_NBR * _C, _C).reshape(_H, _DH, _C) * (1.0 / _NBR)
    bstack = jnp.mean(WO_b, axis=0).reshape(1, _C)

    y = pl.pallas_call(
        _outproj_body,
        grid=(nmt,),
        in_specs=[
            pl.BlockSpec((_H, _QBLK, _DH), lambda i: (0, i, 0)),
            pl.BlockSpec((_H, _DH, _C), lambda i: (0, 0, 0)),
            pl.BlockSpec((1, _C), lambda i: (0, 0)),
        ],
        out_specs=pl.BlockSpec((_QBLK, _C), lambda i: (i, 0)),
        out_shape=jax.ShapeDtypeStruct((_T, _C), _F32),
    )(ctx, wstack, bstack)

    return y.reshape(1, _T, _C)
